# Initial kernel scaffold; baseline (speedup 1.0000x reference)
import jax
import jax.numpy as jnp
from jax.experimental import pallas as pl


def kernel(net, inp, corr, flow, ii, jj, kk, params):
    raise NotImplementedError("write your pallas kernel here")


# speedup vs baseline: 1.2674x; 1.2674x over previous
import functools

import jax
import jax.numpy as jnp
from jax.experimental import pallas as pl
from jax.experimental.pallas import tpu as pltpu

F32 = jnp.float32
_INTERPRET = False

DIM = 384
NK = 2048
NIJ = 64 * 64
BE = 1024


def _mm(x, w):
    return jnp.dot(x, w, preferred_element_type=F32)


def _ln(x, g, b, eps=1e-3):
    mu = jnp.mean(x, axis=-1, keepdims=True)
    xc = x - mu
    var = jnp.mean(xc * xc, axis=-1, keepdims=True)
    return xc * jax.lax.rsqrt(var + eps) * g + b


def _full(shape):
    return pl.BlockSpec(shape, lambda i: (0, 0))


def _rows(width):
    return pl.BlockSpec((BE, width), lambda i: (i, 0))


def _stage_a_body(corr_r, net_r, inp_r, wc0, bc0, wc1, bc1, cg, cb, wc2, bc2,
                  ng, nb, out_r):
    c = jnp.maximum(_mm(corr_r[...], wc0[...]) + bc0[...], 0.0)
    c = _mm(c, wc1[...]) + bc1[...]
    c = jnp.maximum(_ln(c, cg[...], cb[...]), 0.0)
    c = _mm(c, wc2[...]) + bc2[...]
    x = net_r[...] + inp_r[...] + c
    out_r[...] = _ln(x, ng[...], nb[...])


def _stage_a(corr, net, inp, p, cdim, e):
    return pl.pallas_call(
        _stage_a_body,
        grid=(e // BE,),
        in_specs=[
            _rows(cdim), _rows(DIM), _rows(DIM),
            _full((cdim, DIM)), _full((1, DIM)),
            _full((DIM, DIM)), _full((1, DIM)),
            _full((1, DIM)), _full((1, DIM)),
            _full((DIM, DIM)), _full((1, DIM)),
            _full((1, DIM)), _full((1, DIM)),
        ],
        out_specs=_rows(DIM),
        out_shape=jax.ShapeDtypeStruct((e, DIM), F32),
        interpret=_INTERPRET,
    )(corr, net, inp,
      p['corr0']['w'], p['corr0']['b'][None],
      p['corr1']['w'], p['corr1']['b'][None],
      p['corr_ln']['g'][None], p['corr_ln']['b'][None],
      p['corr2']['w'], p['corr2']['b'][None],
      p['norm']['g'][None], p['norm']['b'][None])


def _stage_c_body(x_r, g_r, m_r, w0, b0, w1, b1, out_r):
    z = m_r[:, :1] * g_r[...]
    h = jnp.maximum(_mm(z, w0[...]) + b0[...], 0.0)
    out_r[...] = x_r[...] + _mm(h, w1[...]) + b1[...]


def _stage_c(x, gathered, mask8, p0, p1, e):
    return pl.pallas_call(
        _stage_c_body,
        grid=(e // BE,),
        in_specs=[
            _rows(DIM), _rows(DIM), _rows(8),
            _full((DIM, DIM)), _full((1, DIM)),
            _full((DIM, DIM)), _full((1, DIM)),
        ],
        out_specs=_rows(DIM),
        out_shape=jax.ShapeDtypeStruct((e, DIM), F32),
        interpret=_INTERPRET,
    )(x, gathered, mask8, p0['w'], p0['b'][None], p1['w'], p1['b'][None])


def _stage_e_body(x_r, g_r, m_r, w0, b0, w1, b1, wg, bg, wf, bf,
                  x2_r, gk_r, fk_r, cmax_r):
    z = m_r[:, :1] * g_r[...]
    h = jnp.maximum(_mm(z, w0[...]) + b0[...], 0.0)
    x2 = x_r[...] + _mm(h, w1[...]) + b1[...]
    x2_r[...] = x2
    gk = _mm(x2, wg[...]) + bg[...]
    fk = _mm(x2, wf[...]) + bf[...]
    gk_r[...] = gk
    fk_r[...] = fk
    bm = jnp.max(gk, axis=0, keepdims=True)
    i = pl.program_id(0)

    @pl.when(i == 0)
    def _():
        cmax_r[...] = bm

    @pl.when(i > 0)
    def _():
        cmax_r[...] = jnp.maximum(cmax_r[...], bm)


def _stage_e(x, gathered, mask8, p0, p1, pg, pf, e):
    return pl.pallas_call(
        _stage_e_body,
        grid=(e // BE,),
        in_specs=[
            _rows(DIM), _rows(DIM), _rows(8),
            _full((DIM, DIM)), _full((1, DIM)),
            _full((DIM, DIM)), _full((1, DIM)),
            _full((DIM, DIM)), _full((1, DIM)),
            _full((DIM, DIM)), _full((1, DIM)),
        ],
        out_specs=[_rows(DIM), _rows(DIM), _rows(DIM),
                   pl.BlockSpec((1, DIM), lambda i: (0, 0))],
        out_shape=[jax.ShapeDtypeStruct((e, DIM), F32),
                   jax.ShapeDtypeStruct((e, DIM), F32),
                   jax.ShapeDtypeStruct((e, DIM), F32),
                   jax.ShapeDtypeStruct((1, DIM), F32)],
        interpret=_INTERPRET,
    )(x, gathered, mask8,
      p0['w'], p0['b'][None], p1['w'], p1['b'][None],
      pg['w'], pg['b'][None], pf['w'], pf['b'][None])


def _stage_g_body(x_r, hg_r, wg, bg, wf, bf, x3_r, gk_r, fk_r, cmax_r):
    x3 = x_r[...] + hg_r[...]
    x3_r[...] = x3
    gk = _mm(x3, wg[...]) + bg[...]
    fk = _mm(x3, wf[...]) + bf[...]
    gk_r[...] = gk
    fk_r[...] = fk
    bm = jnp.max(gk, axis=0, keepdims=True)
    i = pl.program_id(0)

    @pl.when(i == 0)
    def _():
        cmax_r[...] = bm

    @pl.when(i > 0)
    def _():
        cmax_r[...] = jnp.maximum(cmax_r[...], bm)


def _stage_g(x, hg, pg, pf, e):
    return pl.pallas_call(
        _stage_g_body,
        grid=(e // BE,),
        in_specs=[
            _rows(DIM), _rows(DIM),
            _full((DIM, DIM)), _full((1, DIM)),
            _full((DIM, DIM)), _full((1, DIM)),
        ],
        out_specs=[_rows(DIM), _rows(DIM), _rows(DIM),
                   pl.BlockSpec((1, DIM), lambda i: (0, 0))],
        out_shape=[jax.ShapeDtypeStruct((e, DIM), F32),
                   jax.ShapeDtypeStruct((e, DIM), F32),
                   jax.ShapeDtypeStruct((e, DIM), F32),
                   jax.ShapeDtypeStruct((1, DIM), F32)],
        interpret=_INTERPRET,
    )(x, hg, pg['w'], pg['b'][None], pf['w'], pf['b'][None])


def _stage_f2_body(g_r, f_r, cmax_r, eu_r):
    e = jnp.exp(g_r[...] - cmax_r[...])
    eu_r[...] = jnp.concatenate([e, f_r[...] * e], axis=-1)


def _stage_f2(g, f, cmax, e):
    return pl.pallas_call(
        _stage_f2_body,
        grid=(e // BE,),
        in_specs=[_rows(DIM), _rows(DIM),
                  pl.BlockSpec((1, DIM), lambda i: (0, 0))],
        out_specs=_rows(2 * DIM),
        out_shape=jax.ShapeDtypeStruct((e, 2 * DIM), F32),
        interpret=_INTERPRET,
    )(g, f, cmax)


def _stage_t_body(t_r, wh, bh, h_r):
    es = t_r[:, :DIM]
    us = t_r[:, DIM:]
    y = us / jnp.maximum(es, 1e-30)
    h_r[...] = _mm(y, wh[...]) + bh[...]


def _stage_t(table, ph, nseg):
    return pl.pallas_call(
        _stage_t_body,
        grid=(1,),
        in_specs=[_full((nseg, 2 * DIM)), _full((DIM, DIM)), _full((1, DIM))],
        out_specs=_full((nseg, DIM)),
        out_shape=jax.ShapeDtypeStruct((nseg, DIM), F32),
        interpret=_INTERPRET,
    )(table, ph['w'], ph['b'][None])


def _stage_h_body(x_r, hg_r, l1g, l1b, gw1, gb1, r0w1, r0b1, r1w1, r1b1,
                  l2g, l2b, gw2, gb2, r0w2, r0b2, r1w2, r1b2, wdw, bdw,
                  net_r, dw_r):
    x = x_r[...] + hg_r[...]
    x = _ln(x, l1g[...], l1b[...])
    gate = jax.nn.sigmoid(_mm(x, gw1[...]) + gb1[...])
    res = _mm(jnp.maximum(_mm(x, r0w1[...]) + r0b1[...], 0.0), r1w1[...]) + r1b1[...]
    x = x + gate * res
    x = _ln(x, l2g[...], l2b[...])
    gate = jax.nn.sigmoid(_mm(x, gw2[...]) + gb2[...])
    res = _mm(jnp.maximum(_mm(x, r0w2[...]) + r0b2[...], 0.0), r1w2[...]) + r1b2[...]
    x = x + gate * res
    net_r[...] = x
    r = jnp.maximum(x, 0.0)
    dw = _mm(r, wdw[...]) + bdw[...]
    li = jax.lax.broadcasted_iota(jnp.int32, dw.shape, 1)
    dw_r[...] = jnp.where(li >= 2, jax.nn.sigmoid(dw), dw)


def _stage_h(x, hg, p, wdw, bdw, e):
    return pl.pallas_call(
        _stage_h_body,
        grid=(e // BE,),
        in_specs=[
            _rows(DIM), _rows(DIM),
            _full((1, DIM)), _full((1, DIM)),
            _full((DIM, DIM)), _full((1, DIM)),
            _full((DIM, DIM)), _full((1, DIM)),
            _full((DIM, DIM)), _full((1, DIM)),
            _full((1, DIM)), _full((1, DIM)),
            _full((DIM, DIM)), _full((1, DIM)),
            _full((DIM, DIM)), _full((1, DIM)),
            _full((DIM, DIM)), _full((1, DIM)),
            _full((DIM, 4)), _full((1, 4)),
        ],
        out_specs=[_rows(DIM), _rows(4)],
        out_shape=[jax.ShapeDtypeStruct((e, DIM), F32),
                   jax.ShapeDtypeStruct((e, 4), F32)],
        interpret=_INTERPRET,
    )(x, hg,
      p['gru_ln1']['g'][None], p['gru_ln1']['b'][None],
      p['gru_gr1']['gate']['w'], p['gru_gr1']['gate']['b'][None],
      p['gru_gr1']['res0']['w'], p['gru_gr1']['res0']['b'][None],
      p['gru_gr1']['res1']['w'], p['gru_gr1']['res1']['b'][None],
      p['gru_ln2']['g'][None], p['gru_ln2']['b'][None],
      p['gru_gr2']['gate']['w'], p['gru_gr2']['gate']['b'][None],
      p['gru_gr2']['res0']['w'], p['gru_gr2']['res0']['b'][None],
      p['gru_gr2']['res1']['w'], p['gru_gr2']['res1']['b'][None],
      wdw, bdw)


def kernel(net, inp, corr, flow, ii, jj, kk, params):
    e = net.shape[1]
    cdim = corr.shape[2]
    p = params
    x_net = net[0]
    x_inp = inp[0]
    x_corr = corr[0]
    kk32 = kk.astype(jnp.int32)
    jj32 = jj.astype(jnp.int32)
    ii32 = ii.astype(jnp.int32)

    m = 66
    key = kk32 * m + jj32
    order = jnp.argsort(key, stable=True)
    skeys = key[order]

    def look(target):
        pos = jnp.searchsorted(skeys, target)
        pos_c = jnp.clip(pos, 0, e - 1)
        found = skeys[pos_c] == target
        return jnp.where(found, order[pos_c], -1)

    ix = look(key - 1)
    jx = look(key + 1)
    mask_ix8 = jnp.broadcast_to((ix >= 0)[:, None], (e, 8)).astype(F32)
    mask_jx8 = jnp.broadcast_to((jx >= 0)[:, None], (e, 8)).astype(F32)
    ixc = jnp.clip(ix, 0, e - 1)
    jxc = jnp.clip(jx, 0, e - 1)
    ij_idx = ii32 * 64 + jj32

    wdw = jnp.concatenate([p['d']['w'], p['w']['w']], axis=1)
    bdw = jnp.concatenate([p['d']['b'], p['w']['b']])[None]

    x0 = _stage_a(x_corr, x_net, x_inp, p, cdim, e)
    x1 = _stage_c(x0, x0[ixc], mask_ix8, p['c1_0'], p['c1_1'], e)
    x2, gk, fk, cmaxk = _stage_e(x1, x1[jxc], mask_jx8, p['c2_0'], p['c2_1'],
                                 p['agg_kk']['g'], p['agg_kk']['f'], e)
    euk = _stage_f2(gk, fk, cmaxk, e)
    tk = jax.ops.segment_sum(euk, kk32, num_segments=NK)
    hk = _stage_t(tk, p['agg_kk']['h'], NK)
    x3, gij, fij, cmaxij = _stage_g(x2, hk[kk32], p['agg_ij']['g'],
                                    p['agg_ij']['f'], e)
    euij = _stage_f2(gij, fij, cmaxij, e)
    tij = jax.ops.segment_sum(euij, ij_idx, num_segments=NIJ)
    hij = _stage_t(tij, p['agg_ij']['h'], NIJ)
    net_out, dw = _stage_h(x3, hij[ij_idx], p, wdw, bdw, e)

    return (net_out[None], dw[None, :, :2], dw[None, :, 2:])


# speedup vs baseline: 1.3136x; 1.0364x over previous
import functools

import jax
import jax.numpy as jnp
from jax import lax
from jax.experimental import pallas as pl
from jax.experimental.pallas import tpu as pltpu
from jax.experimental.pallas import tpu_sc as plsc

F32 = jnp.float32
_INTERPRET = False

DIM = 384
NK = 2048
NIJ = 64 * 64
BE = 1024

NC = 2
NS = 16
NW = NC * NS


def _sc_gather(table, idx, d):
    e_rows = idx.shape[0]
    per_w = e_rows // NW
    ch = 128
    nch = per_w // ch
    mesh = plsc.VectorSubcoreMesh(core_axis_name="c", subcore_axis_name="s")

    @functools.partial(
        pl.kernel,
        out_type=jax.ShapeDtypeStruct((e_rows, d), F32),
        mesh=mesh,
        scratch_types=[
            pltpu.VMEM((ch,), jnp.int32),
            pltpu.VMEM((ch, d), F32),
            pltpu.SemaphoreType.DMA,
        ],
    )
    def k(table_hbm, idx_hbm, out_hbm, idx_v, rows_v, sem):
        wid = lax.axis_index("s") * NC + lax.axis_index("c")
        base = wid * per_w

        def body(i, carry):
            off = base + i * ch
            pltpu.sync_copy(idx_hbm.at[pl.ds(off, ch)], idx_v)
            pltpu.async_copy(table_hbm.at[idx_v], rows_v, sem).wait()
            pltpu.sync_copy(rows_v, out_hbm.at[pl.ds(off, ch)])
            return carry

        lax.fori_loop(0, nch, body, 0)

    return k(table, idx)


def _sc_scatter_add(eu, idx, zeros, nseg):
    e_rows = idx.shape[0]
    per_t = e_rows // NS
    ch = 128
    nch = per_t // ch
    zrows = nseg // NS
    mesh = plsc.VectorSubcoreMesh(core_axis_name="c", subcore_axis_name="s")

    @functools.partial(
        pl.kernel,
        out_type=jax.ShapeDtypeStruct((nseg, 2 * DIM), F32),
        mesh=mesh,
        scratch_types=[
            pltpu.VMEM((ch,), jnp.int32),
            pltpu.VMEM((ch, DIM), F32),
            pltpu.SemaphoreType.DMA,
        ],
    )
    def k(eu_hbm, idx_hbm, zeros_hbm, out_hbm, idx_v, rows_v, sem):
        c = lax.axis_index("c")
        s = lax.axis_index("s")
        col0 = c * DIM
        pltpu.sync_copy(zeros_hbm,
                        out_hbm.at[pl.ds(s * zrows, zrows), pl.ds(col0, DIM)])
        plsc.subcore_barrier()

        def body(i, carry):
            off = s * per_t + i * ch
            pltpu.sync_copy(idx_hbm.at[pl.ds(off, ch)], idx_v)
            pltpu.sync_copy(eu_hbm.at[pl.ds(off, ch), pl.ds(col0, DIM)], rows_v)
            pltpu.sync_copy(rows_v, out_hbm.at[idx_v, pl.ds(col0, DIM)], add=True)
            return carry

        lax.fori_loop(0, nch, body, 0)

    return k(eu, idx, zeros)


def _mm(x, w):
    return jnp.dot(x, w, preferred_element_type=F32)


def _ln(x, g, b, eps=1e-3):
    mu = jnp.mean(x, axis=-1, keepdims=True)
    xc = x - mu
    var = jnp.mean(xc * xc, axis=-1, keepdims=True)
    return xc * jax.lax.rsqrt(var + eps) * g + b


def _full(shape):
    return pl.BlockSpec(shape, lambda i: (0, 0))


def _rows(width):
    return pl.BlockSpec((BE, width), lambda i: (i, 0))


def _stage_a_body(corr_r, net_r, inp_r, wc0, bc0, wc1, bc1, cg, cb, wc2, bc2,
                  ng, nb, out_r):
    c = jnp.maximum(_mm(corr_r[...], wc0[...]) + bc0[...], 0.0)
    c = _mm(c, wc1[...]) + bc1[...]
    c = jnp.maximum(_ln(c, cg[...], cb[...]), 0.0)
    c = _mm(c, wc2[...]) + bc2[...]
    x = net_r[...] + inp_r[...] + c
    out_r[...] = _ln(x, ng[...], nb[...])


def _stage_a(corr, net, inp, p, cdim, e):
    return pl.pallas_call(
        _stage_a_body,
        grid=(e // BE,),
        in_specs=[
            _rows(cdim), _rows(DIM), _rows(DIM),
            _full((cdim, DIM)), _full((1, DIM)),
            _full((DIM, DIM)), _full((1, DIM)),
            _full((1, DIM)), _full((1, DIM)),
            _full((DIM, DIM)), _full((1, DIM)),
            _full((1, DIM)), _full((1, DIM)),
        ],
        out_specs=_rows(DIM),
        out_shape=jax.ShapeDtypeStruct((e, DIM), F32),
        interpret=_INTERPRET,
    )(corr, net, inp,
      p['corr0']['w'], p['corr0']['b'][None],
      p['corr1']['w'], p['corr1']['b'][None],
      p['corr_ln']['g'][None], p['corr_ln']['b'][None],
      p['corr2']['w'], p['corr2']['b'][None],
      p['norm']['g'][None], p['norm']['b'][None])


def _stage_c_body(x_r, g_r, m_r, w0, b0, w1, b1, out_r):
    z = m_r[:, :1] * g_r[...]
    h = jnp.maximum(_mm(z, w0[...]) + b0[...], 0.0)
    out_r[...] = x_r[...] + _mm(h, w1[...]) + b1[...]


def _stage_c(x, gathered, mask8, p0, p1, e):
    return pl.pallas_call(
        _stage_c_body,
        grid=(e // BE,),
        in_specs=[
            _rows(DIM), _rows(DIM), _rows(8),
            _full((DIM, DIM)), _full((1, DIM)),
            _full((DIM, DIM)), _full((1, DIM)),
        ],
        out_specs=_rows(DIM),
        out_shape=jax.ShapeDtypeStruct((e, DIM), F32),
        interpret=_INTERPRET,
    )(x, gathered, mask8, p0['w'], p0['b'][None], p1['w'], p1['b'][None])


def _stage_e_body(x_r, g_r, m_r, w0, b0, w1, b1, wg, bg, wf, bf,
                  x2_r, gk_r, fk_r, cmax_r):
    z = m_r[:, :1] * g_r[...]
    h = jnp.maximum(_mm(z, w0[...]) + b0[...], 0.0)
    x2 = x_r[...] + _mm(h, w1[...]) + b1[...]
    x2_r[...] = x2
    gk = _mm(x2, wg[...]) + bg[...]
    fk = _mm(x2, wf[...]) + bf[...]
    gk_r[...] = gk
    fk_r[...] = fk
    bm = jnp.max(gk, axis=0, keepdims=True)
    i = pl.program_id(0)

    @pl.when(i == 0)
    def _():
        cmax_r[...] = bm

    @pl.when(i > 0)
    def _():
        cmax_r[...] = jnp.maximum(cmax_r[...], bm)


def _stage_e(x, gathered, mask8, p0, p1, pg, pf, e):
    return pl.pallas_call(
        _stage_e_body,
        grid=(e // BE,),
        in_specs=[
            _rows(DIM), _rows(DIM), _rows(8),
            _full((DIM, DIM)), _full((1, DIM)),
            _full((DIM, DIM)), _full((1, DIM)),
            _full((DIM, DIM)), _full((1, DIM)),
            _full((DIM, DIM)), _full((1, DIM)),
        ],
        out_specs=[_rows(DIM), _rows(DIM), _rows(DIM),
                   pl.BlockSpec((1, DIM), lambda i: (0, 0))],
        out_shape=[jax.ShapeDtypeStruct((e, DIM), F32),
                   jax.ShapeDtypeStruct((e, DIM), F32),
                   jax.ShapeDtypeStruct((e, DIM), F32),
                   jax.ShapeDtypeStruct((1, DIM), F32)],
        interpret=_INTERPRET,
    )(x, gathered, mask8,
      p0['w'], p0['b'][None], p1['w'], p1['b'][None],
      pg['w'], pg['b'][None], pf['w'], pf['b'][None])


def _stage_g_body(x_r, hg_r, wg, bg, wf, bf, x3_r, gk_r, fk_r, cmax_r):
    x3 = x_r[...] + hg_r[...]
    x3_r[...] = x3
    gk = _mm(x3, wg[...]) + bg[...]
    fk = _mm(x3, wf[...]) + bf[...]
    gk_r[...] = gk
    fk_r[...] = fk
    bm = jnp.max(gk, axis=0, keepdims=True)
    i = pl.program_id(0)

    @pl.when(i == 0)
    def _():
        cmax_r[...] = bm

    @pl.when(i > 0)
    def _():
        cmax_r[...] = jnp.maximum(cmax_r[...], bm)


def _stage_g(x, hg, pg, pf, e):
    return pl.pallas_call(
        _stage_g_body,
        grid=(e // BE,),
        in_specs=[
            _rows(DIM), _rows(DIM),
            _full((DIM, DIM)), _full((1, DIM)),
            _full((DIM, DIM)), _full((1, DIM)),
        ],
        out_specs=[_rows(DIM), _rows(DIM), _rows(DIM),
                   pl.BlockSpec((1, DIM), lambda i: (0, 0))],
        out_shape=[jax.ShapeDtypeStruct((e, DIM), F32),
                   jax.ShapeDtypeStruct((e, DIM), F32),
                   jax.ShapeDtypeStruct((e, DIM), F32),
                   jax.ShapeDtypeStruct((1, DIM), F32)],
        interpret=_INTERPRET,
    )(x, hg, pg['w'], pg['b'][None], pf['w'], pf['b'][None])


def _stage_f2_body(g_r, f_r, cmax_r, eu_r):
    e = jnp.exp(g_r[...] - cmax_r[...])
    eu_r[...] = jnp.concatenate([e, f_r[...] * e], axis=-1)


def _stage_f2(g, f, cmax, e):
    return pl.pallas_call(
        _stage_f2_body,
        grid=(e // BE,),
        in_specs=[_rows(DIM), _rows(DIM),
                  pl.BlockSpec((1, DIM), lambda i: (0, 0))],
        out_specs=_rows(2 * DIM),
        out_shape=jax.ShapeDtypeStruct((e, 2 * DIM), F32),
        interpret=_INTERPRET,
    )(g, f, cmax)


def _stage_t_body(t_r, wh, bh, h_r):
    es = t_r[:, :DIM]
    us = t_r[:, DIM:]
    y = us / jnp.maximum(es, 1e-30)
    h_r[...] = _mm(y, wh[...]) + bh[...]


def _stage_t(table, ph, nseg):
    return pl.pallas_call(
        _stage_t_body,
        grid=(1,),
        in_specs=[_full((nseg, 2 * DIM)), _full((DIM, DIM)), _full((1, DIM))],
        out_specs=_full((nseg, DIM)),
        out_shape=jax.ShapeDtypeStruct((nseg, DIM), F32),
        interpret=_INTERPRET,
    )(table, ph['w'], ph['b'][None])


def _stage_h_body(x_r, hg_r, l1g, l1b, gw1, gb1, r0w1, r0b1, r1w1, r1b1,
                  l2g, l2b, gw2, gb2, r0w2, r0b2, r1w2, r1b2, wdw, bdw,
                  net_r, dw_r):
    x = x_r[...] + hg_r[...]
    x = _ln(x, l1g[...], l1b[...])
    gate = jax.nn.sigmoid(_mm(x, gw1[...]) + gb1[...])
    res = _mm(jnp.maximum(_mm(x, r0w1[...]) + r0b1[...], 0.0), r1w1[...]) + r1b1[...]
    x = x + gate * res
    x = _ln(x, l2g[...], l2b[...])
    gate = jax.nn.sigmoid(_mm(x, gw2[...]) + gb2[...])
    res = _mm(jnp.maximum(_mm(x, r0w2[...]) + r0b2[...], 0.0), r1w2[...]) + r1b2[...]
    x = x + gate * res
    net_r[...] = x
    r = jnp.maximum(x, 0.0)
    dw = _mm(r, wdw[...]) + bdw[...]
    li = jax.lax.broadcasted_iota(jnp.int32, dw.shape, 1)
    dw_r[...] = jnp.where(li >= 2, jax.nn.sigmoid(dw), dw)


def _stage_h(x, hg, p, wdw, bdw, e):
    return pl.pallas_call(
        _stage_h_body,
        grid=(e // BE,),
        in_specs=[
            _rows(DIM), _rows(DIM),
            _full((1, DIM)), _full((1, DIM)),
            _full((DIM, DIM)), _full((1, DIM)),
            _full((DIM, DIM)), _full((1, DIM)),
            _full((DIM, DIM)), _full((1, DIM)),
            _full((1, DIM)), _full((1, DIM)),
            _full((DIM, DIM)), _full((1, DIM)),
            _full((DIM, DIM)), _full((1, DIM)),
            _full((DIM, DIM)), _full((1, DIM)),
            _full((DIM, 4)), _full((1, 4)),
        ],
        out_specs=[_rows(DIM), _rows(4)],
        out_shape=[jax.ShapeDtypeStruct((e, DIM), F32),
                   jax.ShapeDtypeStruct((e, 4), F32)],
        interpret=_INTERPRET,
    )(x, hg,
      p['gru_ln1']['g'][None], p['gru_ln1']['b'][None],
      p['gru_gr1']['gate']['w'], p['gru_gr1']['gate']['b'][None],
      p['gru_gr1']['res0']['w'], p['gru_gr1']['res0']['b'][None],
      p['gru_gr1']['res1']['w'], p['gru_gr1']['res1']['b'][None],
      p['gru_ln2']['g'][None], p['gru_ln2']['b'][None],
      p['gru_gr2']['gate']['w'], p['gru_gr2']['gate']['b'][None],
      p['gru_gr2']['res0']['w'], p['gru_gr2']['res0']['b'][None],
      p['gru_gr2']['res1']['w'], p['gru_gr2']['res1']['b'][None],
      wdw, bdw)


def kernel(net, inp, corr, flow, ii, jj, kk, params):
    e = net.shape[1]
    cdim = corr.shape[2]
    p = params
    x_net = net[0]
    x_inp = inp[0]
    x_corr = corr[0]
    kk32 = kk.astype(jnp.int32)
    jj32 = jj.astype(jnp.int32)
    ii32 = ii.astype(jnp.int32)

    m = 66
    key = kk32 * m + jj32
    order = jnp.argsort(key, stable=True)
    skeys = key[order]

    def look(target):
        pos = jnp.searchsorted(skeys, target)
        pos_c = jnp.clip(pos, 0, e - 1)
        found = skeys[pos_c] == target
        return jnp.where(found, order[pos_c], -1)

    ix = look(key - 1)
    jx = look(key + 1)
    mask_ix8 = jnp.broadcast_to((ix >= 0)[:, None], (e, 8)).astype(F32)
    mask_jx8 = jnp.broadcast_to((jx >= 0)[:, None], (e, 8)).astype(F32)
    ixc = jnp.clip(ix, 0, e - 1)
    jxc = jnp.clip(jx, 0, e - 1)
    ij_idx = ii32 * 64 + jj32

    wdw = jnp.concatenate([p['d']['w'], p['w']['w']], axis=1)
    bdw = jnp.concatenate([p['d']['b'], p['w']['b']])[None]

    zk = jnp.zeros((NK // NS, DIM), F32)
    zij = jnp.zeros((NIJ // NS, DIM), F32)

    x0 = _stage_a(x_corr, x_net, x_inp, p, cdim, e)
    x1 = _stage_c(x0, _sc_gather(x0, ixc, DIM), mask_ix8,
                  p['c1_0'], p['c1_1'], e)
    x2, gk, fk, cmaxk = _stage_e(x1, _sc_gather(x1, jxc, DIM), mask_jx8,
                                 p['c2_0'], p['c2_1'],
                                 p['agg_kk']['g'], p['agg_kk']['f'], e)
    euk = _stage_f2(gk, fk, cmaxk, e)
    tk = jax.ops.segment_sum(euk, kk32, num_segments=NK)
    hk = _stage_t(tk, p['agg_kk']['h'], NK)
    x3, gij, fij, cmaxij = _stage_g(x2, _sc_gather(hk, kk32, DIM),
                                    p['agg_ij']['g'], p['agg_ij']['f'], e)
    euij = _stage_f2(gij, fij, cmaxij, e)
    tij = jax.ops.segment_sum(euij, ij_idx, num_segments=NIJ)
    hij = _stage_t(tij, p['agg_ij']['h'], NIJ)
    net_out, dw = _stage_h(x3, _sc_gather(hij, ij_idx, DIM), p, wdw, bdw, e)

    return (net_out[None], dw[None, :, :2], dw[None, :, 2:])


# speedup vs baseline: 1.3180x; 1.0033x over previous
import functools

import jax
import jax.numpy as jnp
from jax import lax
from jax.experimental import pallas as pl
from jax.experimental.pallas import tpu as pltpu
from jax.experimental.pallas import tpu_sc as plsc

F32 = jnp.float32
_INTERPRET = False

DIM = 384
NK = 2048
NIJ = 64 * 64
BE = 1024

NC = 2
NS = 16
NW = NC * NS


def _sc_gather(table, idx, d, dtype=F32):
    e_rows = idx.shape[0]
    per_w = e_rows // NW
    ch = 128
    nch = per_w // ch
    mesh = plsc.VectorSubcoreMesh(core_axis_name="c", subcore_axis_name="s")

    @functools.partial(
        pl.kernel,
        out_type=jax.ShapeDtypeStruct((e_rows, d), dtype),
        mesh=mesh,
        scratch_types=[
            pltpu.VMEM((ch,), jnp.int32),
            pltpu.VMEM((ch, d), dtype),
            pltpu.SemaphoreType.DMA,
        ],
    )
    def k(table_hbm, idx_hbm, out_hbm, idx_v, rows_v, sem):
        wid = lax.axis_index("s") * NC + lax.axis_index("c")
        base = wid * per_w

        def body(i, carry):
            off = base + i * ch
            pltpu.sync_copy(idx_hbm.at[pl.ds(off, ch)], idx_v)
            pltpu.async_copy(table_hbm.at[idx_v], rows_v, sem).wait()
            pltpu.sync_copy(rows_v, out_hbm.at[pl.ds(off, ch)])
            return carry

        lax.fori_loop(0, nch, body, 0)

    return k(table, idx)


HD = DIM // 2
PW = 256


def _pack_bf16(x):
    a = x[:, :HD]
    b = x[:, HD:]
    au = jax.lax.bitcast_convert_type(a.astype(jnp.bfloat16), jnp.uint16)
    bu = jax.lax.bitcast_convert_type(b.astype(jnp.bfloat16), jnp.uint16)
    u = (au.astype(jnp.uint32) << 16) | bu.astype(jnp.uint32)
    p = jax.lax.bitcast_convert_type(u, jnp.int32)
    return jnp.pad(p, ((0, 0), (0, PW - HD)))


def _unpack_bf16(p):
    u = jax.lax.bitcast_convert_type(p[:, :HD], jnp.uint32)
    hi = (u >> 16).astype(jnp.uint16)
    lo = (u & jnp.uint32(0xFFFF)).astype(jnp.uint16)
    a = jax.lax.bitcast_convert_type(hi, jnp.bfloat16).astype(F32)
    b = jax.lax.bitcast_convert_type(lo, jnp.bfloat16).astype(F32)
    return jnp.concatenate([a, b], axis=-1)


def _sc_scatter_add(eu, idx, zeros, nseg):
    e_rows = idx.shape[0]
    per_t = e_rows // NS
    ch = 128
    nch = per_t // ch
    zrows = nseg // NS
    mesh = plsc.VectorSubcoreMesh(core_axis_name="c", subcore_axis_name="s")

    @functools.partial(
        pl.kernel,
        out_type=jax.ShapeDtypeStruct((nseg, 2 * DIM), F32),
        mesh=mesh,
        scratch_types=[
            pltpu.VMEM((ch,), jnp.int32),
            pltpu.VMEM((ch, DIM), F32),
            pltpu.SemaphoreType.DMA,
        ],
    )
    def k(eu_hbm, idx_hbm, zeros_hbm, out_hbm, idx_v, rows_v, sem):
        c = lax.axis_index("c")
        s = lax.axis_index("s")
        col0 = c * DIM
        pltpu.sync_copy(zeros_hbm,
                        out_hbm.at[pl.ds(s * zrows, zrows), pl.ds(col0, DIM)])
        plsc.subcore_barrier()

        def body(i, carry):
            off = s * per_t + i * ch
            pltpu.sync_copy(idx_hbm.at[pl.ds(off, ch)], idx_v)
            pltpu.sync_copy(eu_hbm.at[pl.ds(off, ch), pl.ds(col0, DIM)], rows_v)
            pltpu.sync_copy(rows_v, out_hbm.at[idx_v, pl.ds(col0, DIM)], add=True)
            return carry

        lax.fori_loop(0, nch, body, 0)

    return k(eu, idx, zeros)


def _mm(x, w):
    return jnp.dot(x, w, preferred_element_type=F32)


def _ln(x, g, b, eps=1e-3):
    mu = jnp.mean(x, axis=-1, keepdims=True)
    xc = x - mu
    var = jnp.mean(xc * xc, axis=-1, keepdims=True)
    return xc * jax.lax.rsqrt(var + eps) * g + b


def _full(shape):
    return pl.BlockSpec(shape, lambda i: (0, 0))


def _rows(width):
    return pl.BlockSpec((BE, width), lambda i: (i, 0))


def _stage_a_body(corr_r, net_r, inp_r, wc0, bc0, wc1, bc1, cg, cb, wc2, bc2,
                  ng, nb, out_r, outp_r):
    c = jnp.maximum(_mm(corr_r[...], wc0[...]) + bc0[...], 0.0)
    c = _mm(c, wc1[...]) + bc1[...]
    c = jnp.maximum(_ln(c, cg[...], cb[...]), 0.0)
    c = _mm(c, wc2[...]) + bc2[...]
    x = net_r[...] + inp_r[...] + c
    x = _ln(x, ng[...], nb[...])
    out_r[...] = x
    outp_r[...] = _pack_bf16(x)


def _stage_a(corr, net, inp, p, cdim, e):
    return pl.pallas_call(
        _stage_a_body,
        grid=(e // BE,),
        in_specs=[
            _rows(cdim), _rows(DIM), _rows(DIM),
            _full((cdim, DIM)), _full((1, DIM)),
            _full((DIM, DIM)), _full((1, DIM)),
            _full((1, DIM)), _full((1, DIM)),
            _full((DIM, DIM)), _full((1, DIM)),
            _full((1, DIM)), _full((1, DIM)),
        ],
        out_specs=[_rows(DIM), _rows(PW)],
        out_shape=[jax.ShapeDtypeStruct((e, DIM), F32),
                   jax.ShapeDtypeStruct((e, PW), jnp.int32)],
        interpret=_INTERPRET,
    )(corr, net, inp,
      p['corr0']['w'], p['corr0']['b'][None],
      p['corr1']['w'], p['corr1']['b'][None],
      p['corr_ln']['g'][None], p['corr_ln']['b'][None],
      p['corr2']['w'], p['corr2']['b'][None],
      p['norm']['g'][None], p['norm']['b'][None])


def _stage_c_body(x_r, g_r, m_r, w0, b0, w1, b1, out_r, outp_r):
    z = m_r[:, :1] * _unpack_bf16(g_r[...])
    h = jnp.maximum(_mm(z, w0[...]) + b0[...], 0.0)
    x = x_r[...] + _mm(h, w1[...]) + b1[...]
    out_r[...] = x
    outp_r[...] = _pack_bf16(x)


def _stage_c(x, gathered, mask8, p0, p1, e):
    return pl.pallas_call(
        _stage_c_body,
        grid=(e // BE,),
        in_specs=[
            _rows(DIM), _rows(PW), _rows(8),
            _full((DIM, DIM)), _full((1, DIM)),
            _full((DIM, DIM)), _full((1, DIM)),
        ],
        out_specs=[_rows(DIM), _rows(PW)],
        out_shape=[jax.ShapeDtypeStruct((e, DIM), F32),
                   jax.ShapeDtypeStruct((e, PW), jnp.int32)],
        interpret=_INTERPRET,
    )(x, gathered, mask8, p0['w'], p0['b'][None], p1['w'], p1['b'][None])


def _stage_e_body(x_r, g_r, m_r, w0, b0, w1, b1, wg, bg, wf, bf,
                  x2_r, gk_r, fk_r, cmax_r):
    z = m_r[:, :1] * _unpack_bf16(g_r[...])
    h = jnp.maximum(_mm(z, w0[...]) + b0[...], 0.0)
    x2 = x_r[...] + _mm(h, w1[...]) + b1[...]
    x2_r[...] = x2
    gk = _mm(x2, wg[...]) + bg[...]
    fk = _mm(x2, wf[...]) + bf[...]
    gk_r[...] = gk
    fk_r[...] = fk
    bm = jnp.max(gk, axis=0, keepdims=True)
    i = pl.program_id(0)

    @pl.when(i == 0)
    def _():
        cmax_r[...] = bm

    @pl.when(i > 0)
    def _():
        cmax_r[...] = jnp.maximum(cmax_r[...], bm)


def _stage_e(x, gathered, mask8, p0, p1, pg, pf, e):
    return pl.pallas_call(
        _stage_e_body,
        grid=(e // BE,),
        in_specs=[
            _rows(DIM), _rows(PW), _rows(8),
            _full((DIM, DIM)), _full((1, DIM)),
            _full((DIM, DIM)), _full((1, DIM)),
            _full((DIM, DIM)), _full((1, DIM)),
            _full((DIM, DIM)), _full((1, DIM)),
        ],
        out_specs=[_rows(DIM), _rows(DIM), _rows(DIM),
                   pl.BlockSpec((1, DIM), lambda i: (0, 0))],
        out_shape=[jax.ShapeDtypeStruct((e, DIM), F32),
                   jax.ShapeDtypeStruct((e, DIM), F32),
                   jax.ShapeDtypeStruct((e, DIM), F32),
                   jax.ShapeDtypeStruct((1, DIM), F32)],
        interpret=_INTERPRET,
    )(x, gathered, mask8,
      p0['w'], p0['b'][None], p1['w'], p1['b'][None],
      pg['w'], pg['b'][None], pf['w'], pf['b'][None])


def _stage_g_body(x_r, hg_r, wg, bg, wf, bf, x3_r, gk_r, fk_r, cmax_r):
    x3 = x_r[...] + hg_r[...]
    x3_r[...] = x3
    gk = _mm(x3, wg[...]) + bg[...]
    fk = _mm(x3, wf[...]) + bf[...]
    gk_r[...] = gk
    fk_r[...] = fk
    bm = jnp.max(gk, axis=0, keepdims=True)
    i = pl.program_id(0)

    @pl.when(i == 0)
    def _():
        cmax_r[...] = bm

    @pl.when(i > 0)
    def _():
        cmax_r[...] = jnp.maximum(cmax_r[...], bm)


def _stage_g(x, hg, pg, pf, e):
    return pl.pallas_call(
        _stage_g_body,
        grid=(e // BE,),
        in_specs=[
            _rows(DIM), _rows(DIM),
            _full((DIM, DIM)), _full((1, DIM)),
            _full((DIM, DIM)), _full((1, DIM)),
        ],
        out_specs=[_rows(DIM), _rows(DIM), _rows(DIM),
                   pl.BlockSpec((1, DIM), lambda i: (0, 0))],
        out_shape=[jax.ShapeDtypeStruct((e, DIM), F32),
                   jax.ShapeDtypeStruct((e, DIM), F32),
                   jax.ShapeDtypeStruct((e, DIM), F32),
                   jax.ShapeDtypeStruct((1, DIM), F32)],
        interpret=_INTERPRET,
    )(x, hg, pg['w'], pg['b'][None], pf['w'], pf['b'][None])


def _stage_f2_body(g_r, f_r, cmax_r, eu_r):
    e = jnp.exp(g_r[...] - cmax_r[...])
    eu_r[...] = jnp.concatenate([e, f_r[...] * e], axis=-1)


def _stage_f2(g, f, cmax, e):
    return pl.pallas_call(
        _stage_f2_body,
        grid=(e // BE,),
        in_specs=[_rows(DIM), _rows(DIM),
                  pl.BlockSpec((1, DIM), lambda i: (0, 0))],
        out_specs=_rows(2 * DIM),
        out_shape=jax.ShapeDtypeStruct((e, 2 * DIM), F32),
        interpret=_INTERPRET,
    )(g, f, cmax)


def _stage_t_body(t_r, wh, bh, h_r):
    es = t_r[:, :DIM]
    us = t_r[:, DIM:]
    y = us / jnp.maximum(es, 1e-30)
    h_r[...] = _mm(y, wh[...]) + bh[...]


def _stage_t(table, ph, nseg):
    return pl.pallas_call(
        _stage_t_body,
        grid=(1,),
        in_specs=[_full((nseg, 2 * DIM)), _full((DIM, DIM)), _full((1, DIM))],
        out_specs=_full((nseg, DIM)),
        out_shape=jax.ShapeDtypeStruct((nseg, DIM), F32),
        interpret=_INTERPRET,
    )(table, ph['w'], ph['b'][None])


def _stage_h_body(x_r, hg_r, l1g, l1b, gw1, gb1, r0w1, r0b1, r1w1, r1b1,
                  l2g, l2b, gw2, gb2, r0w2, r0b2, r1w2, r1b2, wdw, bdw,
                  net_r, dw_r):
    x = x_r[...] + hg_r[...]
    x = _ln(x, l1g[...], l1b[...])
    gate = jax.nn.sigmoid(_mm(x, gw1[...]) + gb1[...])
    res = _mm(jnp.maximum(_mm(x, r0w1[...]) + r0b1[...], 0.0), r1w1[...]) + r1b1[...]
    x = x + gate * res
    x = _ln(x, l2g[...], l2b[...])
    gate = jax.nn.sigmoid(_mm(x, gw2[...]) + gb2[...])
    res = _mm(jnp.maximum(_mm(x, r0w2[...]) + r0b2[...], 0.0), r1w2[...]) + r1b2[...]
    x = x + gate * res
    net_r[...] = x
    r = jnp.maximum(x, 0.0)
    dw = _mm(r, wdw[...]) + bdw[...]
    li = jax.lax.broadcasted_iota(jnp.int32, dw.shape, 1)
    dw_r[...] = jnp.where(li >= 2, jax.nn.sigmoid(dw), dw)


def _stage_h(x, hg, p, wdw, bdw, e):
    return pl.pallas_call(
        _stage_h_body,
        grid=(e // BE,),
        in_specs=[
            _rows(DIM), _rows(DIM),
            _full((1, DIM)), _full((1, DIM)),
            _full((DIM, DIM)), _full((1, DIM)),
            _full((DIM, DIM)), _full((1, DIM)),
            _full((DIM, DIM)), _full((1, DIM)),
            _full((1, DIM)), _full((1, DIM)),
            _full((DIM, DIM)), _full((1, DIM)),
            _full((DIM, DIM)), _full((1, DIM)),
            _full((DIM, DIM)), _full((1, DIM)),
            _full((DIM, 4)), _full((1, 4)),
        ],
        out_specs=[_rows(DIM), _rows(4)],
        out_shape=[jax.ShapeDtypeStruct((e, DIM), F32),
                   jax.ShapeDtypeStruct((e, 4), F32)],
        interpret=_INTERPRET,
    )(x, hg,
      p['gru_ln1']['g'][None], p['gru_ln1']['b'][None],
      p['gru_gr1']['gate']['w'], p['gru_gr1']['gate']['b'][None],
      p['gru_gr1']['res0']['w'], p['gru_gr1']['res0']['b'][None],
      p['gru_gr1']['res1']['w'], p['gru_gr1']['res1']['b'][None],
      p['gru_ln2']['g'][None], p['gru_ln2']['b'][None],
      p['gru_gr2']['gate']['w'], p['gru_gr2']['gate']['b'][None],
      p['gru_gr2']['res0']['w'], p['gru_gr2']['res0']['b'][None],
      p['gru_gr2']['res1']['w'], p['gru_gr2']['res1']['b'][None],
      wdw, bdw)


def kernel(net, inp, corr, flow, ii, jj, kk, params):
    e = net.shape[1]
    cdim = corr.shape[2]
    p = params
    x_net = net[0]
    x_inp = inp[0]
    x_corr = corr[0]
    kk32 = kk.astype(jnp.int32)
    jj32 = jj.astype(jnp.int32)
    ii32 = ii.astype(jnp.int32)

    m = 66
    key = kk32 * m + jj32
    order = jnp.argsort(key, stable=True)
    skeys = key[order]

    def look(target):
        pos = jnp.searchsorted(skeys, target)
        pos_c = jnp.clip(pos, 0, e - 1)
        found = skeys[pos_c] == target
        return jnp.where(found, order[pos_c], -1)

    ix = look(key - 1)
    jx = look(key + 1)
    mask_ix8 = jnp.broadcast_to((ix >= 0)[:, None], (e, 8)).astype(F32)
    mask_jx8 = jnp.broadcast_to((jx >= 0)[:, None], (e, 8)).astype(F32)
    ixc = jnp.clip(ix, 0, e - 1)
    jxc = jnp.clip(jx, 0, e - 1)
    ij_idx = ii32 * 64 + jj32

    wdw = jnp.concatenate([p['d']['w'], p['w']['w']], axis=1)
    bdw = jnp.concatenate([p['d']['b'], p['w']['b']])[None]

    zk = jnp.zeros((NK // NS, DIM), F32)
    zij = jnp.zeros((NIJ // NS, DIM), F32)

    x0, x0p = _stage_a(x_corr, x_net, x_inp, p, cdim, e)
    x1, x1p = _stage_c(x0, _sc_gather(x0p, ixc, PW, jnp.int32), mask_ix8,
                       p['c1_0'], p['c1_1'], e)
    x2, gk, fk, cmaxk = _stage_e(x1, _sc_gather(x1p, jxc, PW, jnp.int32),
                                 mask_jx8, p['c2_0'], p['c2_1'],
                                 p['agg_kk']['g'], p['agg_kk']['f'], e)
    euk = _stage_f2(gk, fk, cmaxk, e)
    tk = jax.ops.segment_sum(euk, kk32, num_segments=NK)
    hk = _stage_t(tk, p['agg_kk']['h'], NK)
    x3, gij, fij, cmaxij = _stage_g(x2, _sc_gather(hk, kk32, DIM),
                                    p['agg_ij']['g'], p['agg_ij']['f'], e)
    euij = _stage_f2(gij, fij, cmaxij, e)
    tij = jax.ops.segment_sum(euij, ij_idx, num_segments=NIJ)
    hij = _stage_t(tij, p['agg_ij']['h'], NIJ)
    net_out, dw = _stage_h(x3, _sc_gather(hij, ij_idx, DIM), p, wdw, bdw, e)

    return (net_out[None], dw[None, :, :2], dw[None, :, 2:])


# speedup vs baseline: 1.3191x; 1.0008x over previous
import functools

import jax
import jax.numpy as jnp
from jax import lax
from jax.experimental import pallas as pl
from jax.experimental.pallas import tpu as pltpu
from jax.experimental.pallas import tpu_sc as plsc

F32 = jnp.float32
_INTERPRET = False

DIM = 384
NK = 2048
NIJ = 64 * 64
BE = 1024

NC = 2
NS = 16
NW = NC * NS


def _sc_gather(table, idx, d, dtype=F32):
    e_rows = idx.shape[0]
    per_w = e_rows // NW
    ch = 128 if d <= 256 else 64
    nch = per_w // ch
    nbuf = 3
    mesh = plsc.VectorSubcoreMesh(core_axis_name="c", subcore_axis_name="s")

    @functools.partial(
        pl.kernel,
        out_type=jax.ShapeDtypeStruct((e_rows, d), dtype),
        mesh=mesh,
        scratch_types=[
            pltpu.VMEM((nbuf, ch), jnp.int32),
            pltpu.VMEM((nbuf, ch, d), dtype),
            pltpu.SemaphoreType.DMA,
        ],
    )
    def k(table_hbm, idx_hbm, out_hbm, idx_v, rows_v, sem):
        wid = lax.axis_index("s") * NC + lax.axis_index("c")
        base = wid * per_w
        cps = {}

        def issue(i):
            slot = i % nbuf
            off = base + i * ch
            pltpu.sync_copy(idx_hbm.at[pl.ds(off, ch)], idx_v.at[slot])
            cps[i] = pltpu.async_copy(table_hbm.at[idx_v.at[slot]],
                                      rows_v.at[slot], sem)

        for i in range(nbuf):
            issue(i)
        for i in range(nch):
            slot = i % nbuf
            cps[i].wait()
            pltpu.sync_copy(rows_v.at[slot],
                            out_hbm.at[pl.ds(base + i * ch, ch)])
            if i + nbuf < nch:
                issue(i + nbuf)

    return k(table, idx)


HD = DIM // 2
PW = 256


def _pack_bf16(x):
    a = x[:, :HD]
    b = x[:, HD:]
    au = jax.lax.bitcast_convert_type(a.astype(jnp.bfloat16), jnp.uint16)
    bu = jax.lax.bitcast_convert_type(b.astype(jnp.bfloat16), jnp.uint16)
    u = (au.astype(jnp.uint32) << 16) | bu.astype(jnp.uint32)
    p = jax.lax.bitcast_convert_type(u, jnp.int32)
    return jnp.pad(p, ((0, 0), (0, PW - HD)))


def _unpack_bf16(p):
    u = jax.lax.bitcast_convert_type(p[:, :HD], jnp.uint32)
    hi = (u >> 16).astype(jnp.uint16)
    lo = (u & jnp.uint32(0xFFFF)).astype(jnp.uint16)
    a = jax.lax.bitcast_convert_type(hi, jnp.bfloat16).astype(F32)
    b = jax.lax.bitcast_convert_type(lo, jnp.bfloat16).astype(F32)
    return jnp.concatenate([a, b], axis=-1)


def _sc_scatter_add(eu, idx, zeros, nseg):
    e_rows = idx.shape[0]
    per_t = e_rows // NS
    ch = 128
    nch = per_t // ch
    zrows = nseg // NS
    mesh = plsc.VectorSubcoreMesh(core_axis_name="c", subcore_axis_name="s")

    @functools.partial(
        pl.kernel,
        out_type=jax.ShapeDtypeStruct((nseg, 2 * DIM), F32),
        mesh=mesh,
        scratch_types=[
            pltpu.VMEM((ch,), jnp.int32),
            pltpu.VMEM((ch, DIM), F32),
            pltpu.SemaphoreType.DMA,
        ],
    )
    def k(eu_hbm, idx_hbm, zeros_hbm, out_hbm, idx_v, rows_v, sem):
        c = lax.axis_index("c")
        s = lax.axis_index("s")
        col0 = c * DIM
        pltpu.sync_copy(zeros_hbm,
                        out_hbm.at[pl.ds(s * zrows, zrows), pl.ds(col0, DIM)])
        plsc.subcore_barrier()

        def body(i, carry):
            off = s * per_t + i * ch
            pltpu.sync_copy(idx_hbm.at[pl.ds(off, ch)], idx_v)
            pltpu.sync_copy(eu_hbm.at[pl.ds(off, ch), pl.ds(col0, DIM)], rows_v)
            pltpu.sync_copy(rows_v, out_hbm.at[idx_v, pl.ds(col0, DIM)], add=True)
            return carry

        lax.fori_loop(0, nch, body, 0)

    return k(eu, idx, zeros)


def _mm(x, w):
    return jnp.dot(x, w, preferred_element_type=F32)


def _ln(x, g, b, eps=1e-3):
    mu = jnp.mean(x, axis=-1, keepdims=True)
    xc = x - mu
    var = jnp.mean(xc * xc, axis=-1, keepdims=True)
    return xc * jax.lax.rsqrt(var + eps) * g + b


def _full(shape):
    return pl.BlockSpec(shape, lambda i: (0, 0))


def _rows(width):
    return pl.BlockSpec((BE, width), lambda i: (i, 0))


def _stage_a_body(corr_r, net_r, inp_r, wc0, bc0, wc1, bc1, cg, cb, wc2, bc2,
                  ng, nb, out_r, outp_r):
    c = jnp.maximum(_mm(corr_r[...], wc0[...]) + bc0[...], 0.0)
    c = _mm(c, wc1[...]) + bc1[...]
    c = jnp.maximum(_ln(c, cg[...], cb[...]), 0.0)
    c = _mm(c, wc2[...]) + bc2[...]
    x = net_r[...] + inp_r[...] + c
    x = _ln(x, ng[...], nb[...])
    out_r[...] = x
    outp_r[...] = _pack_bf16(x)


def _stage_a(corr, net, inp, p, cdim, e):
    return pl.pallas_call(
        _stage_a_body,
        grid=(e // BE,),
        in_specs=[
            _rows(cdim), _rows(DIM), _rows(DIM),
            _full((cdim, DIM)), _full((1, DIM)),
            _full((DIM, DIM)), _full((1, DIM)),
            _full((1, DIM)), _full((1, DIM)),
            _full((DIM, DIM)), _full((1, DIM)),
            _full((1, DIM)), _full((1, DIM)),
        ],
        out_specs=[_rows(DIM), _rows(PW)],
        out_shape=[jax.ShapeDtypeStruct((e, DIM), F32),
                   jax.ShapeDtypeStruct((e, PW), jnp.int32)],
        interpret=_INTERPRET,
    )(corr, net, inp,
      p['corr0']['w'], p['corr0']['b'][None],
      p['corr1']['w'], p['corr1']['b'][None],
      p['corr_ln']['g'][None], p['corr_ln']['b'][None],
      p['corr2']['w'], p['corr2']['b'][None],
      p['norm']['g'][None], p['norm']['b'][None])


def _stage_c_body(x_r, g_r, m_r, w0, b0, w1, b1, out_r, outp_r):
    z = m_r[:, :1] * _unpack_bf16(g_r[...])
    h = jnp.maximum(_mm(z, w0[...]) + b0[...], 0.0)
    x = x_r[...] + _mm(h, w1[...]) + b1[...]
    out_r[...] = x
    outp_r[...] = _pack_bf16(x)


def _stage_c(x, gathered, mask8, p0, p1, e):
    return pl.pallas_call(
        _stage_c_body,
        grid=(e // BE,),
        in_specs=[
            _rows(DIM), _rows(PW), _rows(8),
            _full((DIM, DIM)), _full((1, DIM)),
            _full((DIM, DIM)), _full((1, DIM)),
        ],
        out_specs=[_rows(DIM), _rows(PW)],
        out_shape=[jax.ShapeDtypeStruct((e, DIM), F32),
                   jax.ShapeDtypeStruct((e, PW), jnp.int32)],
        interpret=_INTERPRET,
    )(x, gathered, mask8, p0['w'], p0['b'][None], p1['w'], p1['b'][None])


def _stage_e_body(x_r, g_r, m_r, w0, b0, w1, b1, wg, bg, wf, bf,
                  x2_r, gk_r, fk_r, cmax_r):
    z = m_r[:, :1] * _unpack_bf16(g_r[...])
    h = jnp.maximum(_mm(z, w0[...]) + b0[...], 0.0)
    x2 = x_r[...] + _mm(h, w1[...]) + b1[...]
    x2_r[...] = x2
    gk = _mm(x2, wg[...]) + bg[...]
    fk = _mm(x2, wf[...]) + bf[...]
    gk_r[...] = gk
    fk_r[...] = fk
    bm = jnp.max(gk, axis=0, keepdims=True)
    i = pl.program_id(0)

    @pl.when(i == 0)
    def _():
        cmax_r[...] = bm

    @pl.when(i > 0)
    def _():
        cmax_r[...] = jnp.maximum(cmax_r[...], bm)


def _stage_e(x, gathered, mask8, p0, p1, pg, pf, e):
    return pl.pallas_call(
        _stage_e_body,
        grid=(e // BE,),
        in_specs=[
            _rows(DIM), _rows(PW), _rows(8),
            _full((DIM, DIM)), _full((1, DIM)),
            _full((DIM, DIM)), _full((1, DIM)),
            _full((DIM, DIM)), _full((1, DIM)),
            _full((DIM, DIM)), _full((1, DIM)),
        ],
        out_specs=[_rows(DIM), _rows(DIM), _rows(DIM),
                   pl.BlockSpec((1, DIM), lambda i: (0, 0))],
        out_shape=[jax.ShapeDtypeStruct((e, DIM), F32),
                   jax.ShapeDtypeStruct((e, DIM), F32),
                   jax.ShapeDtypeStruct((e, DIM), F32),
                   jax.ShapeDtypeStruct((1, DIM), F32)],
        interpret=_INTERPRET,
    )(x, gathered, mask8,
      p0['w'], p0['b'][None], p1['w'], p1['b'][None],
      pg['w'], pg['b'][None], pf['w'], pf['b'][None])


def _stage_g_body(x_r, hg_r, wg, bg, wf, bf, x3_r, gk_r, fk_r, cmax_r):
    x3 = x_r[...] + hg_r[...]
    x3_r[...] = x3
    gk = _mm(x3, wg[...]) + bg[...]
    fk = _mm(x3, wf[...]) + bf[...]
    gk_r[...] = gk
    fk_r[...] = fk
    bm = jnp.max(gk, axis=0, keepdims=True)
    i = pl.program_id(0)

    @pl.when(i == 0)
    def _():
        cmax_r[...] = bm

    @pl.when(i > 0)
    def _():
        cmax_r[...] = jnp.maximum(cmax_r[...], bm)


def _stage_g(x, hg, pg, pf, e):
    return pl.pallas_call(
        _stage_g_body,
        grid=(e // BE,),
        in_specs=[
            _rows(DIM), _rows(DIM),
            _full((DIM, DIM)), _full((1, DIM)),
            _full((DIM, DIM)), _full((1, DIM)),
        ],
        out_specs=[_rows(DIM), _rows(DIM), _rows(DIM),
                   pl.BlockSpec((1, DIM), lambda i: (0, 0))],
        out_shape=[jax.ShapeDtypeStruct((e, DIM), F32),
                   jax.ShapeDtypeStruct((e, DIM), F32),
                   jax.ShapeDtypeStruct((e, DIM), F32),
                   jax.ShapeDtypeStruct((1, DIM), F32)],
        interpret=_INTERPRET,
    )(x, hg, pg['w'], pg['b'][None], pf['w'], pf['b'][None])


def _stage_f2_body(g_r, f_r, cmax_r, eu_r):
    e = jnp.exp(g_r[...] - cmax_r[...])
    eu_r[...] = jnp.concatenate([e, f_r[...] * e], axis=-1)


def _stage_f2(g, f, cmax, e):
    return pl.pallas_call(
        _stage_f2_body,
        grid=(e // BE,),
        in_specs=[_rows(DIM), _rows(DIM),
                  pl.BlockSpec((1, DIM), lambda i: (0, 0))],
        out_specs=_rows(2 * DIM),
        out_shape=jax.ShapeDtypeStruct((e, 2 * DIM), F32),
        interpret=_INTERPRET,
    )(g, f, cmax)


def _stage_t_body(t_r, wh, bh, h_r):
    es = t_r[:, :DIM]
    us = t_r[:, DIM:]
    y = us / jnp.maximum(es, 1e-30)
    h_r[...] = _mm(y, wh[...]) + bh[...]


def _stage_t(table, ph, nseg):
    return pl.pallas_call(
        _stage_t_body,
        grid=(1,),
        in_specs=[_full((nseg, 2 * DIM)), _full((DIM, DIM)), _full((1, DIM))],
        out_specs=_full((nseg, DIM)),
        out_shape=jax.ShapeDtypeStruct((nseg, DIM), F32),
        interpret=_INTERPRET,
    )(table, ph['w'], ph['b'][None])


def _stage_h_body(x_r, hg_r, l1g, l1b, gw1, gb1, r0w1, r0b1, r1w1, r1b1,
                  l2g, l2b, gw2, gb2, r0w2, r0b2, r1w2, r1b2, wdw, bdw,
                  net_r, dw_r):
    x = x_r[...] + hg_r[...]
    x = _ln(x, l1g[...], l1b[...])
    gate = jax.nn.sigmoid(_mm(x, gw1[...]) + gb1[...])
    res = _mm(jnp.maximum(_mm(x, r0w1[...]) + r0b1[...], 0.0), r1w1[...]) + r1b1[...]
    x = x + gate * res
    x = _ln(x, l2g[...], l2b[...])
    gate = jax.nn.sigmoid(_mm(x, gw2[...]) + gb2[...])
    res = _mm(jnp.maximum(_mm(x, r0w2[...]) + r0b2[...], 0.0), r1w2[...]) + r1b2[...]
    x = x + gate * res
    net_r[...] = x
    r = jnp.maximum(x, 0.0)
    dw = _mm(r, wdw[...]) + bdw[...]
    li = jax.lax.broadcasted_iota(jnp.int32, dw.shape, 1)
    dw_r[...] = jnp.where(li >= 2, jax.nn.sigmoid(dw), dw)


def _stage_h(x, hg, p, wdw, bdw, e):
    return pl.pallas_call(
        _stage_h_body,
        grid=(e // BE,),
        in_specs=[
            _rows(DIM), _rows(DIM),
            _full((1, DIM)), _full((1, DIM)),
            _full((DIM, DIM)), _full((1, DIM)),
            _full((DIM, DIM)), _full((1, DIM)),
            _full((DIM, DIM)), _full((1, DIM)),
            _full((1, DIM)), _full((1, DIM)),
            _full((DIM, DIM)), _full((1, DIM)),
            _full((DIM, DIM)), _full((1, DIM)),
            _full((DIM, DIM)), _full((1, DIM)),
            _full((DIM, 4)), _full((1, 4)),
        ],
        out_specs=[_rows(DIM), _rows(4)],
        out_shape=[jax.ShapeDtypeStruct((e, DIM), F32),
                   jax.ShapeDtypeStruct((e, 4), F32)],
        interpret=_INTERPRET,
    )(x, hg,
      p['gru_ln1']['g'][None], p['gru_ln1']['b'][None],
      p['gru_gr1']['gate']['w'], p['gru_gr1']['gate']['b'][None],
      p['gru_gr1']['res0']['w'], p['gru_gr1']['res0']['b'][None],
      p['gru_gr1']['res1']['w'], p['gru_gr1']['res1']['b'][None],
      p['gru_ln2']['g'][None], p['gru_ln2']['b'][None],
      p['gru_gr2']['gate']['w'], p['gru_gr2']['gate']['b'][None],
      p['gru_gr2']['res0']['w'], p['gru_gr2']['res0']['b'][None],
      p['gru_gr2']['res1']['w'], p['gru_gr2']['res1']['b'][None],
      wdw, bdw)


def kernel(net, inp, corr, flow, ii, jj, kk, params):
    e = net.shape[1]
    cdim = corr.shape[2]
    p = params
    x_net = net[0]
    x_inp = inp[0]
    x_corr = corr[0]
    kk32 = kk.astype(jnp.int32)
    jj32 = jj.astype(jnp.int32)
    ii32 = ii.astype(jnp.int32)

    m = 66
    key = kk32 * m + jj32
    order = jnp.argsort(key, stable=True)
    skeys = key[order]

    def look(target):
        pos = jnp.searchsorted(skeys, target)
        pos_c = jnp.clip(pos, 0, e - 1)
        found = skeys[pos_c] == target
        return jnp.where(found, order[pos_c], -1)

    ix = look(key - 1)
    jx = look(key + 1)
    mask_ix8 = jnp.broadcast_to((ix >= 0)[:, None], (e, 8)).astype(F32)
    mask_jx8 = jnp.broadcast_to((jx >= 0)[:, None], (e, 8)).astype(F32)
    ixc = jnp.clip(ix, 0, e - 1)
    jxc = jnp.clip(jx, 0, e - 1)
    ij_idx = ii32 * 64 + jj32

    wdw = jnp.concatenate([p['d']['w'], p['w']['w']], axis=1)
    bdw = jnp.concatenate([p['d']['b'], p['w']['b']])[None]

    zk = jnp.zeros((NK // NS, DIM), F32)
    zij = jnp.zeros((NIJ // NS, DIM), F32)

    x0, x0p = _stage_a(x_corr, x_net, x_inp, p, cdim, e)
    x1, x1p = _stage_c(x0, _sc_gather(x0p, ixc, PW, jnp.int32), mask_ix8,
                       p['c1_0'], p['c1_1'], e)
    x2, gk, fk, cmaxk = _stage_e(x1, _sc_gather(x1p, jxc, PW, jnp.int32),
                                 mask_jx8, p['c2_0'], p['c2_1'],
                                 p['agg_kk']['g'], p['agg_kk']['f'], e)
    euk = _stage_f2(gk, fk, cmaxk, e)
    tk = jax.ops.segment_sum(euk, kk32, num_segments=NK)
    hk = _stage_t(tk, p['agg_kk']['h'], NK)
    x3, gij, fij, cmaxij = _stage_g(x2, _sc_gather(hk, kk32, DIM),
                                    p['agg_ij']['g'], p['agg_ij']['f'], e)
    euij = _stage_f2(gij, fij, cmaxij, e)
    tij = jax.ops.segment_sum(euij, ij_idx, num_segments=NIJ)
    hij = _stage_t(tij, p['agg_ij']['h'], NIJ)
    net_out, dw = _stage_h(x3, _sc_gather(hij, ij_idx, DIM), p, wdw, bdw, e)

    return (net_out[None], dw[None, :, :2], dw[None, :, 2:])
